# baseline (speedup 1.0000x reference)
import jax
import jax.numpy as jnp
from jax import lax
from jax.experimental import pallas as pl
from jax.experimental.pallas import tpu as pltpu
from jax.experimental.pallas import tpu_sc as plsc

M = 1_000_000
D = 64
B = 16384
TBL = 1 << 20
NC, NS = 2, 16
NW = NC * NS
SB = B // NW
IB = B // NS
ROUNDS = 3
FILL = 8192
REG = TBL // NS
NSEM = 8
HB = 256
RPS = HB // NSEM


def _body(mem_hbm, idx_hbm, val_hbm, samp_hbm, out_hbm,
          tbl_sh, fill_v, idxs_v, jv_v, cur_v, tgt_v,
          samp_v, p_v, rows_v, *sems):
    c = lax.axis_index("c")
    s = lax.axis_index("s")
    wid = s * NC + c
    ii16 = lax.iota(jnp.int32, 16)
    neg1 = jnp.full((16,), -1, jnp.int32)

    with jax.named_scope("ph_memset"):
        def _fill(i, _):
            fill_v[pl.ds(i * 16, 16)] = neg1
            return _
        lax.fori_loop(0, FILL // 16, _fill, 0)
        for b in range(REG // FILL):
            pltpu.sync_copy(fill_v,
                            tbl_sh.at[pl.ds(s * REG + b * FILL, FILL)])
        pltpu.sync_copy(idx_hbm.at[pl.ds(s * IB, IB)], idxs_v)

        def _jv(g, _):
            jv_v[pl.ds(g * 16, 16)] = s * IB + g * 16 + ii16
            return _
        lax.fori_loop(0, IB // 16, _jv, 0)
        plsc.subcore_barrier()

    with jax.named_scope("ph_rounds"):
        dump_cell = jnp.full((16,), M, jnp.int32) + wid
        for r in range(ROUNDS):
            if r == 0:
                src = idxs_v
            else:
                pltpu.sync_copy(tbl_sh.at[idxs_v], cur_v)

                def _cmp(g, _):
                    cu = cur_v[pl.ds(g * 16, 16)]
                    jj = jv_v[pl.ds(g * 16, 16)]
                    ix = idxs_v[pl.ds(g * 16, 16)]
                    tgt_v[pl.ds(g * 16, 16)] = jnp.where(
                        cu < jj, ix, dump_cell)
                    return _
                lax.fori_loop(0, IB // 16, _cmp, 0)
                src = tgt_v
            pltpu.sync_copy(jv_v, tbl_sh.at[src])
            plsc.subcore_barrier()

    with jax.named_scope("ph_b"):
        pltpu.sync_copy(samp_hbm.at[pl.ds(wid * SB, SB)], samp_v)
        pltpu.sync_copy(tbl_sh.at[samp_v], p_v)
        for h in range(SB // HB):
            for b in range(NSEM):
                sem = sems[b]

                def _grp(g, _):
                    base = h * HB + b * RPS + g * 16
                    sv16 = samp_v[pl.ds(base, 16)]
                    pv16 = p_v[pl.ds(base, 16)]
                    for l in range(16):
                        sv = sv16[l]
                        pv = pv16[l]
                        o = b * RPS + g * 16 + l

                        @pl.when(pv >= 0)
                        def _hit():
                            pltpu.async_copy(val_hbm.at[pv], rows_v.at[o],
                                             sem)

                        @pl.when(pv < 0)
                        def _miss():
                            pltpu.async_copy(mem_hbm.at[sv], rows_v.at[o],
                                             sem)

                    return _
                lax.fori_loop(0, RPS // 16, _grp, 0)
            for b in range(NSEM):
                pltpu.make_async_copy(
                    mem_hbm.at[pl.ds(0, RPS)],
                    rows_v.at[pl.ds(b * RPS, RPS)], sems[b]).wait()
            pltpu.sync_copy(rows_v,
                            out_hbm.at[pl.ds(wid * SB + h * HB, HB)])


def _build():
    mesh = plsc.VectorSubcoreMesh(core_axis_name="c", subcore_axis_name="s")
    return pl.kernel(
        _body,
        out_type=jax.ShapeDtypeStruct((B, D), jnp.float32),
        mesh=mesh,
        compiler_params=pltpu.CompilerParams(use_tc_tiling_on_sc=False),
        scratch_types=[
            pltpu.VMEM_SHARED((TBL,), jnp.int32),
            pltpu.VMEM((FILL,), jnp.int32),
            pltpu.VMEM((IB,), jnp.int32),
            pltpu.VMEM((IB,), jnp.int32),
            pltpu.VMEM((IB,), jnp.int32),
            pltpu.VMEM((IB,), jnp.int32),
            pltpu.VMEM((SB,), jnp.int32),
            pltpu.VMEM((SB,), jnp.int32),
            pltpu.VMEM((HB, D), jnp.float32),
        ] + [pltpu.SemaphoreType.DMA] * NSEM,
    )


_sc_kernel = _build()


def kernel(mem, idx, val, sample_idx):
    return _sc_kernel(mem, idx, val, sample_idx)


# speedup vs baseline: 1.6370x; 1.6370x over previous
import jax
import jax.numpy as jnp
from jax import lax
from jax.experimental import pallas as pl
from jax.experimental.pallas import tpu as pltpu
from jax.experimental.pallas import tpu_sc as plsc

M = 1_000_000
D = 64
B = 16384
TBL = 1 << 20
NC, NS = 2, 16
NW = NC * NS
SB = B // NW
IB = B // NS
ROUNDS = 3
FILL = 4096
REG = TBL // NS
NSEM = 8
HB = 128
RPS = HB // NSEM


def _body(mem_hbm, idx_hbm, val_hbm, samp_hbm, out_hbm,
          tbl_sh, fill_v, idxs_v, jv_v, cur_v, tgt_v,
          samp_v, p_v, rows_v, *sems):
    c = lax.axis_index("c")
    s = lax.axis_index("s")
    wid = s * NC + c
    ii16 = lax.iota(jnp.int32, 16)
    neg1 = jnp.full((16,), -1, jnp.int32)

    with jax.named_scope("ph_memset"):
        def _fill(i, _):
            fill_v[pl.ds(i * 16, 16)] = neg1
            return _
        lax.fori_loop(0, FILL // 16, _fill, 0)
        for b in range(REG // FILL):
            pltpu.sync_copy(fill_v,
                            tbl_sh.at[pl.ds(s * REG + b * FILL, FILL)])
        pltpu.sync_copy(idx_hbm.at[pl.ds(s * IB, IB)], idxs_v)

        def _jv(g, _):
            jv_v[pl.ds(g * 16, 16)] = s * IB + g * 16 + ii16
            return _
        lax.fori_loop(0, IB // 16, _jv, 0)
        plsc.subcore_barrier()

    with jax.named_scope("ph_rounds"):
        dump_cell = jnp.full((16,), M, jnp.int32) + wid
        for r in range(ROUNDS):
            if r == 0:
                src = idxs_v
            else:
                pltpu.sync_copy(tbl_sh.at[idxs_v], cur_v)

                def _cmp(g, _):
                    cu = cur_v[pl.ds(g * 16, 16)]
                    jj = jv_v[pl.ds(g * 16, 16)]
                    ix = idxs_v[pl.ds(g * 16, 16)]
                    tgt_v[pl.ds(g * 16, 16)] = jnp.where(
                        cu < jj, ix, dump_cell)
                    return _
                lax.fori_loop(0, IB // 16, _cmp, 0)
                src = tgt_v
            pltpu.sync_copy(jv_v, tbl_sh.at[src])
            plsc.subcore_barrier()

    with jax.named_scope("ph_b"):
        pltpu.sync_copy(samp_hbm.at[pl.ds(wid * SB, SB)], samp_v)
        pltpu.sync_copy(tbl_sh.at[samp_v], p_v)
        for h in range(SB // HB):
            for b in range(NSEM):
                sem = sems[b]

                def _grp(g, _):
                    base = h * HB + b * RPS + g * 16
                    sv16 = samp_v[pl.ds(base, 16)]
                    pv16 = p_v[pl.ds(base, 16)]
                    sk16 = ((sv16 >> 3) << 4) + (sv16 & 7) * 2
                    pc16 = jnp.maximum(pv16, 0)
                    pk16 = ((pc16 >> 3) << 4) + (pc16 & 7) * 2
                    for l in range(16):
                        pv = pv16[l]
                        sk = sk16[l]
                        pk = pk16[l]
                        i = b * RPS + g * 16 + l
                        o = (i >> 3) * 16 + (i & 7) * 2

                        @pl.when(pv >= 0)
                        def _hit():
                            pltpu.async_copy(val_hbm.at[pk], rows_v.at[o],
                                             sem)

                        @pl.when(pv < 0)
                        def _miss():
                            pltpu.async_copy(mem_hbm.at[sk], rows_v.at[o],
                                             sem)

                    return _
                lax.fori_loop(0, RPS // 16, _grp, 0)
            for b in range(NSEM):
                pltpu.make_async_copy(
                    mem_hbm.at[pl.ds(0, RPS)],
                    rows_v.at[pl.ds(b * RPS * 2, RPS)], sems[b]).wait()
            pltpu.sync_copy(rows_v,
                            out_hbm.at[pl.ds((wid * SB + h * HB) * 2,
                                             2 * HB)])


def _build():
    mesh = plsc.VectorSubcoreMesh(core_axis_name="c", subcore_axis_name="s")
    return pl.kernel(
        _body,
        out_type=jax.ShapeDtypeStruct((B, D), jnp.float32),
        mesh=mesh,
        compiler_params=pltpu.CompilerParams(disable_bounds_checks=True),
        scratch_types=[
            pltpu.VMEM_SHARED((TBL,), jnp.int32),
            pltpu.VMEM((FILL,), jnp.int32),
            pltpu.VMEM((IB,), jnp.int32),
            pltpu.VMEM((IB,), jnp.int32),
            pltpu.VMEM((IB,), jnp.int32),
            pltpu.VMEM((IB,), jnp.int32),
            pltpu.VMEM((SB,), jnp.int32),
            pltpu.VMEM((SB,), jnp.int32),
            pltpu.VMEM((2 * HB, D), jnp.float32),
        ] + [pltpu.SemaphoreType.DMA] * NSEM,
    )


_sc_kernel = _build()


def kernel(mem, idx, val, sample_idx):
    return _sc_kernel(mem, idx, val, sample_idx)
